# Initial kernel scaffold; baseline (speedup 1.0000x reference)
import jax
import jax.numpy as jnp
from jax.experimental import pallas as pl


def kernel(h, weight, bias, mean_scale):
    raise NotImplementedError("write your pallas kernel here")


# speedup vs baseline: 5.2213x; 5.2213x over previous
import jax
import jax.numpy as jnp
from jax.experimental import pallas as pl

_GROUP = 12500
_COL_BLK = 128


def _graphnorm_block(h_ref, w_ref, b_ref, ms_ref, o_ref):
    x = h_ref[...]
    n = x.shape[0]
    mean = jnp.sum(x, axis=0, keepdims=True) * (1.0 / n)
    centered = x - mean * ms_ref[...]
    var = jnp.sum(centered * centered, axis=0, keepdims=True) * (1.0 / n)
    inv_std = jax.lax.rsqrt(var + 1e-6)
    o_ref[...] = w_ref[...] * centered * inv_std + b_ref[...]


def kernel(h, weight, bias, mean_scale):
    n_rows, d = h.shape
    batch = n_rows // _GROUP
    hf = h.astype(jnp.float32).reshape(batch, _GROUP, d)
    w2 = weight.astype(jnp.float32).reshape(1, d)
    b2 = bias.astype(jnp.float32).reshape(1, d)
    ms2 = mean_scale.astype(jnp.float32).reshape(1, d)

    out = pl.pallas_call(
        _graphnorm_block,
        grid=(batch, d // _COL_BLK),
        in_specs=[
            pl.BlockSpec((None, _GROUP, _COL_BLK), lambda i, j: (i, 0, j)),
            pl.BlockSpec((1, _COL_BLK), lambda i, j: (0, j)),
            pl.BlockSpec((1, _COL_BLK), lambda i, j: (0, j)),
            pl.BlockSpec((1, _COL_BLK), lambda i, j: (0, j)),
        ],
        out_specs=pl.BlockSpec((None, _GROUP, _COL_BLK), lambda i, j: (i, 0, j)),
        out_shape=jax.ShapeDtypeStruct((batch, _GROUP, d), jnp.float32),
    )(hf, w2, b2, ms2)

    return out.reshape(n_rows, d).astype(h.dtype)
